# Initial kernel scaffold; baseline (speedup 1.0000x reference)
import jax
import jax.numpy as jnp
from jax.experimental import pallas as pl


def kernel(h, edge_index, edge_type, W):
    raise NotImplementedError("write your pallas kernel here")


# speedup vs baseline: 1.1540x; 1.1540x over previous
import functools
import jax
import jax.numpy as jnp
from jax import lax
from jax.experimental import pallas as pl
from jax.experimental.pallas import tpu as pltpu
from jax.experimental.pallas import tpu_sc as plsc

N_NODES = 10000
N_EDGES = 320000
D = 128
N_RELS = 10

NC = 2
NS = 16
L = 16
NW = NC * NS

B = 80
PER_W = N_EDGES // NW
N_CHUNKS = PER_W // B


def _sc_body(h_hbm, src_hbm, dst_hbm, et_hbm, w_hbm, out_hbm,
             idx_u, idx_v, et_v, rows_u, rows_v, w_v, out_v, sem_u, sem_v):
    wid = lax.axis_index("s") * NC + lax.axis_index("c")
    base0 = wid * PER_W

    pltpu.sync_copy(w_hbm, w_v)

    lanes = lax.iota(jnp.int32, L)

    def chunk_body(c, _):
        base = base0 + c * B
        pltpu.sync_copy(src_hbm.at[pl.ds(base, B)], idx_u)
        pltpu.sync_copy(dst_hbm.at[pl.ds(base, B)], idx_v)
        pltpu.sync_copy(et_hbm.at[pl.ds(base, B)], et_v)
        cu = pltpu.async_copy(h_hbm.at[idx_u], rows_u, sem_u)
        cv = pltpu.async_copy(h_hbm.at[idx_v], rows_v, sem_v)
        cu.wait()
        cv.wait()
        def group_body(g, _):
            eid = lanes + g * L
            r = et_v[pl.ds(g * L, L)]
            acc = jnp.zeros((L,), jnp.float32)
            for d in range(D):
                col = jnp.full((L,), d, jnp.int32)
                u = plsc.load_gather(rows_u, [eid, col])
                v = plsc.load_gather(rows_v, [eid, col])
                w = plsc.load_gather(w_v, [r, col])
                acc = acc + (u * v) * w
            out_v[pl.ds(g * L, L)] = 1.0 / (1.0 + jnp.exp(-acc))
            return _

        lax.fori_loop(0, B // L, group_body, None)
        pltpu.sync_copy(out_v, out_hbm.at[pl.ds(base, B)])
        return _

    lax.fori_loop(0, N_CHUNKS, chunk_body, None)


@jax.jit
def _dist_mult_sc(h, src, dst, et, W):
    mesh = plsc.VectorSubcoreMesh(core_axis_name="c", subcore_axis_name="s",
                                  num_cores=NC, num_subcores=NS)
    return pl.kernel(
        _sc_body,
        out_type=jax.ShapeDtypeStruct((N_EDGES,), jnp.float32),
        mesh=mesh,
        scratch_types=[
            pltpu.VMEM((B,), jnp.int32),
            pltpu.VMEM((B,), jnp.int32),
            pltpu.VMEM((B,), jnp.int32),
            pltpu.VMEM((B, D), jnp.float32),
            pltpu.VMEM((B, D), jnp.float32),
            pltpu.VMEM((N_RELS, D), jnp.float32),
            pltpu.VMEM((B,), jnp.float32),
            pltpu.SemaphoreType.DMA,
            pltpu.SemaphoreType.DMA,
        ],
        compiler_params=pltpu.CompilerParams(needs_layout_passes=False),
    )(h, src, dst, et, W)


def kernel(h, edge_index, edge_type, W):
    src = edge_index[0].astype(jnp.int32)
    dst = edge_index[1].astype(jnp.int32)
    et = edge_type.astype(jnp.int32)
    return _dist_mult_sc(h, src, dst, et, W)


# speedup vs baseline: 1.3792x; 1.1951x over previous
import jax
import jax.numpy as jnp
from jax import lax
from jax.experimental import pallas as pl
from jax.experimental.pallas import tpu as pltpu
from jax.experimental.pallas import tpu_sc as plsc

N_NODES = 10000
N_EDGES = 320000
D = 128
N_RELS = 10

NC = 2
NS = 16
L = 16
NW = NC * NS

B = 80
PER_W = N_EDGES // NW
N_CHUNKS = PER_W // B


def _sc_body(h_hbm, src_hbm, dst_hbm, et_hbm, w_hbm, out_hbm,
             isrc, idst, iet, out_all, rows_u, rows_v, w_v,
             su0, su1, sv0, sv1):
    wid = lax.axis_index("s") * NC + lax.axis_index("c")
    base0 = wid * PER_W

    pltpu.sync_copy(w_hbm, w_v)
    pltpu.sync_copy(src_hbm.at[pl.ds(base0, PER_W)], isrc)
    pltpu.sync_copy(dst_hbm.at[pl.ds(base0, PER_W)], idst)
    pltpu.sync_copy(et_hbm.at[pl.ds(base0, PER_W)], iet)

    sus = [su0, su1]
    svs = [sv0, sv1]
    lanes = lax.iota(jnp.int32, L)

    def fire(c, b):
        pltpu.async_copy(h_hbm.at[isrc.at[pl.ds(c * B, B)]], rows_u.at[b], sus[b])
        pltpu.async_copy(h_hbm.at[idst.at[pl.ds(c * B, B)]], rows_v.at[b], svs[b])

    def wait(c, b):
        pltpu.make_async_copy(h_hbm.at[isrc.at[pl.ds(c * B, B)]],
                              rows_u.at[b], sus[b]).wait()
        pltpu.make_async_copy(h_hbm.at[idst.at[pl.ds(c * B, B)]],
                              rows_v.at[b], svs[b]).wait()

    def compute(c, b):
        ru = rows_u.at[b]
        rv = rows_v.at[b]

        def group_body(g, _):
            eid = lanes + g * L
            r = iet[pl.ds(c * B + g * L, L)]
            acc = jnp.zeros((L,), jnp.float32)
            for d in range(D):
                col = jnp.full((L,), d, jnp.int32)
                u = plsc.load_gather(ru, [eid, col])
                v = plsc.load_gather(rv, [eid, col])
                w = plsc.load_gather(w_v, [r, col])
                acc = acc + (u * v) * w
            out_all[pl.ds(c * B + g * L, L)] = 1.0 / (1.0 + jnp.exp(-acc))
            return _

        lax.fori_loop(0, B // L, group_body, None)

    fire(0, 0)
    fire(1, 1)

    def pair_body(c2, _):
        for b in range(2):
            c = c2 * 2 + b
            wait(c, b)
            compute(c, b)

            @pl.when(c + 2 < N_CHUNKS)
            def _f():
                fire(c + 2, b)
        return _

    lax.fori_loop(0, (N_CHUNKS - 1) // 2, pair_body, None)
    wait(N_CHUNKS - 1, 0)
    compute(N_CHUNKS - 1, 0)

    pltpu.sync_copy(out_all, out_hbm.at[pl.ds(base0, PER_W)])


@jax.jit
def _dist_mult_sc(h, src, dst, et, W):
    mesh = plsc.VectorSubcoreMesh(core_axis_name="c", subcore_axis_name="s",
                                  num_cores=NC, num_subcores=NS)
    return pl.kernel(
        _sc_body,
        out_type=jax.ShapeDtypeStruct((N_EDGES,), jnp.float32),
        mesh=mesh,
        scratch_types=[
            pltpu.VMEM((PER_W,), jnp.int32),
            pltpu.VMEM((PER_W,), jnp.int32),
            pltpu.VMEM((PER_W,), jnp.int32),
            pltpu.VMEM((PER_W,), jnp.float32),
            pltpu.VMEM((2, B, D), jnp.float32),
            pltpu.VMEM((2, B, D), jnp.float32),
            pltpu.VMEM((N_RELS, D), jnp.float32),
            pltpu.SemaphoreType.DMA,
            pltpu.SemaphoreType.DMA,
            pltpu.SemaphoreType.DMA,
            pltpu.SemaphoreType.DMA,
        ],
        compiler_params=pltpu.CompilerParams(needs_layout_passes=False),
    )(h, src, dst, et, W)


def kernel(h, edge_index, edge_type, W):
    src = edge_index[0].astype(jnp.int32)
    dst = edge_index[1].astype(jnp.int32)
    et = edge_type.astype(jnp.int32)
    return _dist_mult_sc(h, src, dst, et, W)
